# Initial kernel scaffold; baseline (speedup 1.0000x reference)
import jax
import jax.numpy as jnp
from jax.experimental import pallas as pl


def kernel(input, edge_index, W, W1, b1, W2, b2, W3, b3, W4, b4):
    raise NotImplementedError("write your pallas kernel here")


# speedup vs baseline: 3.5940x; 3.5940x over previous
import functools

import jax
import jax.numpy as jnp
import numpy as np
from jax import lax
from jax.experimental import pallas as pl
from jax.experimental.pallas import tpu as pltpu
from jax.experimental.pallas import tpu_sc as plsc

N = 2048
E = 32768
IN_FEATURES = 512
OUT_FEATURES = 64
OUT_CHANNELS = 4
ORDER = 8
ALPHA = 0.2
M = ORDER + 1

CB = 256
RT = 256
NB = N // CB
MM_PREC = lax.Precision.DEFAULT

_jj = np.arange(M, dtype=np.float32)
LAM9 = (np.cos(np.pi * (_jj + 0.5) / M) + 1.0).astype(np.float32).reshape(M, 1)
COSMAT = np.cos(np.pi * _jj[:, None] * (_jj[None, :] + 0.5) / M).astype(np.float32)


_EDGES_PER_TILE = E // 16
_ROWS_PER_PHASE = 512
_WORDS_PER_PHASE = _ROWS_PER_PHASE * N
_TILE_SHARE = _WORDS_PER_PHASE // 16


def _sc_count_body(edge_hbm, out_hbm, rows_v, cols_v, idxb, valb, zbuf, shared):
    cid = lax.axis_index("c")
    sid = lax.axis_index("s")
    base_e = sid * _EDGES_PER_TILE
    pltpu.sync_copy(edge_hbm.at[0, pl.ds(base_e, _EDGES_PER_TILE)], rows_v)
    pltpu.sync_copy(edge_hbm.at[1, pl.ds(base_e, _EDGES_PER_TILE)], cols_v)

    def _zero_zbuf(i, carry):
        zbuf[pl.ds(i * 16, 16)] = jnp.zeros((16,), jnp.float32)
        return carry

    lax.fori_loop(0, zbuf.shape[0] // 16, _zero_zbuf, 0)

    for phase in range(2):
        base_row = cid * (2 * _ROWS_PER_PHASE) + phase * _ROWS_PER_PHASE
        for z in range(_TILE_SHARE // zbuf.shape[0]):
            pltpu.sync_copy(
                zbuf,
                shared.at[pl.ds(sid * _TILE_SHARE + z * zbuf.shape[0],
                                zbuf.shape[0])],
            )
        plsc.subcore_barrier()
        for j2 in range(_EDGES_PER_TILE // 128):
            for l in range(8):
                off = j2 * 128 + l * 16
                r = rows_v[pl.ds(off, 16)]
                cc = cols_v[pl.ds(off, 16)]
                rel = r - base_row
                inw = (rel >= 0) & (rel < _ROWS_PER_PHASE)
                idx = (rel & (_ROWS_PER_PHASE - 1)) * N + cc
                val = jnp.where(inw, jnp.full((16,), 1.0, jnp.float32),
                                jnp.zeros((16,), jnp.float32))
                idxb[j2, pl.ds(l * 16, 16)] = idx
                valb[j2, pl.ds(l * 16, 16)] = val
            pltpu.sync_copy(valb.at[j2], shared.at[idxb.at[j2]], add=True)
        plsc.subcore_barrier()
        out_off = base_row * N + sid * _TILE_SHARE
        pltpu.sync_copy(shared.at[pl.ds(sid * _TILE_SHARE, _TILE_SHARE)],
                        out_hbm.at[pl.ds(out_off, _TILE_SHARE)])
        plsc.subcore_barrier()


def _sc_count(edge_index):
    mesh = plsc.VectorSubcoreMesh(core_axis_name="c", subcore_axis_name="s")
    kern = functools.partial(
        pl.kernel,
        out_type=jax.ShapeDtypeStruct((N * N,), jnp.float32),
        mesh=mesh,
        scratch_types=[
            pltpu.VMEM((_EDGES_PER_TILE,), jnp.int32),
            pltpu.VMEM((_EDGES_PER_TILE,), jnp.int32),
            pltpu.VMEM((_EDGES_PER_TILE // 128, 128), jnp.int32),
            pltpu.VMEM((_EDGES_PER_TILE // 128, 128), jnp.float32),
            pltpu.VMEM((4096,), jnp.float32),
            pltpu.VMEM_SHARED((_WORDS_PER_PHASE,), jnp.float32),
        ],
    )(_sc_count_body)
    return kern(edge_index)


def _deg_body(acount_ref, dinv_ref):
    deg = jnp.sum(acount_ref[...], axis=1, keepdims=True)
    dinv_ref[...] = jnp.where(deg > 0, lax.rsqrt(deg), 0.0)


def _deg(acount):
    return pl.pallas_call(
        _deg_body,
        grid=(8,),
        in_specs=[pl.BlockSpec((N // 8, N), lambda i: (i, 0))],
        out_specs=pl.BlockSpec((N // 8, 1), lambda i: (i, 0)),
        out_shape=jax.ShapeDtypeStruct((N, 1), jnp.float32),
    )(acount)


def _scale_body(acount_ref, dinv_ref, input_ref, w_ref, lam_ref, cos_ref,
                w1_ref, b1_ref, w2_ref, b2_ref, w3_ref, b3_ref, w4_ref, b4_ref,
                aneg_ref, h_ref, c_ref):
    i = pl.program_id(0)
    dall = dinv_ref[...]
    drow = dinv_ref[pl.ds(i * (N // 8), N // 8), :]
    aneg_ref[...] = -(drow * dall.T) * acount_ref[...]
    h_ref[...] = jnp.dot(input_ref[...], w_ref[...],
                         preferred_element_type=jnp.float32,
                         precision=MM_PREC)

    @pl.when(i == 0)
    def _coeffs():
        x = lam_ref[...]
        x = jax.nn.relu(x * w1_ref[...] + b1_ref[...])
        x = jax.nn.relu(jnp.dot(x, w2_ref[...], precision=MM_PREC) + b2_ref[...])
        x = jax.nn.relu(jnp.dot(x, w3_ref[...], precision=MM_PREC) + b3_ref[...])
        x = jax.nn.relu(jnp.dot(x, w4_ref[...], precision=MM_PREC) + b4_ref[...])
        c_ref[...] = (2.0 / M) * jnp.dot(cos_ref[...], x, precision=MM_PREC)


def _scale(acount, dinv, input, W, W1, b1, W2, b2, W3, b3, W4, b4):
    def full(shape):
        return pl.BlockSpec(shape, lambda i, _s=shape: tuple(0 for _ in _s))

    return pl.pallas_call(
        _scale_body,
        grid=(8,),
        in_specs=[
            pl.BlockSpec((N // 8, N), lambda i: (i, 0)),
            full((N, 1)),
            pl.BlockSpec((N // 8, IN_FEATURES), lambda i: (i, 0)),
            full((IN_FEATURES, OUT_FEATURES)),
            full((M, 1)),
            full((M, M)),
            full((1, 32)), full((1, 32)),
            full((32, 64)), full((1, 64)),
            full((64, 32)), full((1, 32)),
            full((32, OUT_CHANNELS)), full((1, OUT_CHANNELS)),
        ],
        out_specs=[
            pl.BlockSpec((N // 8, N), lambda i: (i, 0)),
            pl.BlockSpec((N // 8, OUT_FEATURES), lambda i: (i, 0)),
            full((M, OUT_CHANNELS)),
        ],
        out_shape=[
            jax.ShapeDtypeStruct((N, N), jnp.float32),
            jax.ShapeDtypeStruct((N, OUT_FEATURES), jnp.float32),
            jax.ShapeDtypeStruct((M, OUT_CHANNELS), jnp.float32),
        ],
    )(acount, dinv, input, W,
      jnp.asarray(LAM9), jnp.asarray(COSMAT),
      W1, b1.reshape(1, -1), W2, b2.reshape(1, -1),
      W3, b3.reshape(1, -1), W4, b4.reshape(1, -1))


def _main_body(aneg_ref, h_ref, c_ref, out_ref, tbuf, hacc, outacc, divacc):
    b = pl.program_id(0)
    k = pl.program_id(1)

    @pl.when((b == 0) & (k == 0))
    def _zero_accs():
        outacc[...] = jnp.zeros_like(outacc)
        divacc[...] = jnp.zeros_like(divacc)

    @pl.when(k == 0)
    def _init():
        rows = lax.broadcasted_iota(jnp.int32, (N, CB), 0)
        cols = lax.broadcasted_iota(jnp.int32, (N, CB), 1) + b * CB
        t0 = (rows == cols).astype(jnp.float32)
        t1 = aneg_ref[:, pl.ds(b * CB, CB)]
        tbuf[0] = t0
        tbuf[1] = t1
        for ch in range(OUT_CHANNELS):
            hacc[ch] = 0.5 * c_ref[0, ch] * t0 + c_ref[1, ch] * t1

    def _step(prev_s, cur_s, new_s):
        cur = tbuf[cur_s]
        for i in range(N // RT):
            sl = pl.ds(i * RT, RT)
            tnew = 2.0 * jnp.dot(aneg_ref[sl, :], cur,
                                 preferred_element_type=jnp.float32,
                                 precision=MM_PREC) - tbuf[prev_s, sl, :]
            tbuf[new_s, sl, :] = tnew
            for ch in range(OUT_CHANNELS):
                hacc[ch, sl, :] = hacc[ch, sl, :] + c_ref[k + 1, ch] * tnew

    for r in range(1, ORDER):
        @pl.when(k == r)
        def _do(r=r):
            _step((r - 1) % 3, r % 3, (r + 1) % 3)

    @pl.when(k == ORDER - 1)
    def _attention():
        hblk = h_ref[pl.ds(b * CB, CB), :]
        for i in range(N // RT):
            sl = pl.ds(i * RT, RT)
            for ch in range(OUT_CHANNELS):
                hc = hacc[ch, sl, :]
                l = jnp.where(hc > 0, hc, ALPHA * hc)
                l = jnp.where(jnp.isnan(l) | (l < 0), -9e15, l)
                att = jnp.minimum(jnp.exp(l), 9e15)
                csl = slice(ch * OUT_FEATURES, (ch + 1) * OUT_FEATURES)
                outacc[sl, csl] = outacc[sl, csl] + jnp.dot(
                    att, hblk, preferred_element_type=jnp.float32,
                    precision=MM_PREC)
                divacc[sl, ch:ch + 1] = (divacc[sl, ch:ch + 1]
                                         + jnp.sum(att, axis=1, keepdims=True))

    @pl.when((b == NB - 1) & (k == ORDER - 1))
    def _finalize():
        for ch in range(OUT_CHANNELS):
            d = divacc[:, ch:ch + 1]
            d = jnp.where(d == 0, 1.0, d)
            x = outacc[:, ch * OUT_FEATURES:(ch + 1) * OUT_FEATURES] / d
            out_ref[:, ch * OUT_FEATURES:(ch + 1) * OUT_FEATURES] = jnp.where(
                x > 0, x, jnp.exp(x) - 1.0)


def _main(aneg, h, c):
    return pl.pallas_call(
        _main_body,
        grid=(NB, ORDER),
        in_specs=[
            pl.BlockSpec((N, N), lambda b, k: (0, 0)),
            pl.BlockSpec((N, OUT_FEATURES), lambda b, k: (0, 0)),
            pl.BlockSpec((M, OUT_CHANNELS), lambda b, k: (0, 0),
                         memory_space=pltpu.SMEM),
        ],
        out_specs=pl.BlockSpec((N, OUT_CHANNELS * OUT_FEATURES),
                               lambda b, k: (0, 0)),
        out_shape=jax.ShapeDtypeStruct((N, OUT_CHANNELS * OUT_FEATURES),
                                       jnp.float32),
        scratch_shapes=[
            pltpu.VMEM((3, N, CB), jnp.float32),
            pltpu.VMEM((OUT_CHANNELS, N, CB), jnp.float32),
            pltpu.VMEM((N, OUT_CHANNELS * OUT_FEATURES), jnp.float32),
            pltpu.VMEM((N, OUT_CHANNELS), jnp.float32),
        ],
    )(aneg, h, c)


def kernel(input, edge_index, W, W1, b1, W2, b2, W3, b3, W4, b4):
    acount = _sc_count(edge_index).reshape(N, N)
    dinv = _deg(acount)
    aneg, h, c = _scale(acount, dinv, input, W, W1, b1, W2, b2, W3, b3, W4, b4)
    return _main(aneg, h, c)
